# baseline (speedup 1.0000x reference)
import functools

import jax
import jax.numpy as jnp
from jax import lax
from jax.experimental import pallas as pl
from jax.experimental.pallas import tpu as pltpu
from jax.experimental.pallas import tpu_sc as plsc

N = 10000
E = 320000
D = 128

NC = 2
NS = 16
NW = NC * NS
EPW = E // NW
K = 40
CHUNKS = EPW // K
N_PAD = 10240
RPT = N_PAD // NS

NX = 4
NE = 6
NIr = 6
G = 12
DG = 2
DI = 4
DH = D // 2

_sc_mesh = plsc.VectorSubcoreMesh(core_axis_name="c", subcore_axis_name="s")

_scratch = []
_scratch += [pltpu.VMEM((K,), jnp.int32)] * NIr
_scratch += [pltpu.VMEM((K,), jnp.int32)] * NIr
_scratch += [pltpu.VMEM((K, DH), jnp.int32)] * NX
_scratch += [pltpu.VMEM((K, D), jnp.float32)] * NE
_scratch += [pltpu.SemaphoreType.DMA] * (2 * NIr)
_scratch += [pltpu.SemaphoreType.DMA] * NX
_scratch += [pltpu.SemaphoreType.DMA] * (2 * NE)
_scratch += [pltpu.VMEM_SHARED((N_PAD, D), jnp.float32)]


@functools.partial(
    pl.kernel,
    mesh=_sc_mesh,
    out_type=jax.ShapeDtypeStruct((NC, N_PAD, D), jnp.float32),
    scratch_types=_scratch,
    compiler_params=pltpu.CompilerParams(use_tc_tiling_on_sc=False),
)
def _sc_aggregate(x_hbm, ei_hbm, ea_hbm, out_hbm, *refs):
    o = 0
    sidx = list(refs[o:o + NIr]); o += NIr
    didx = list(refs[o:o + NIr]); o += NIr
    xr = list(refs[o:o + NX]); o += NX
    er = list(refs[o:o + NE]); o += NE
    isems = list(refs[o:o + NIr]); o += NIr
    isemd = list(refs[o:o + NIr]); o += NIr
    gsem = list(refs[o:o + NX]); o += NX
    esem = list(refs[o:o + NE]); o += NE
    ssem = list(refs[o:o + NE]); o += NE
    acc = refs[o]

    c = lax.axis_index("c")
    s = lax.axis_index("s")
    wid = c * NS + s
    base = wid * EPW

    def idx_start(bi6, off):
        pltpu.async_copy(ei_hbm.at[pl.ds(off, K)], sidx[bi6], isems[bi6])
        pltpu.async_copy(ei_hbm.at[pl.ds(E + off, K)], didx[bi6], isemd[bi6])

    def gather_start(b4, b6, bi6, off):
        pltpu.make_async_copy(ei_hbm.at[pl.ds(0, K)], sidx[bi6],
                              isems[bi6]).wait()
        pltpu.make_async_copy(ei_hbm.at[pl.ds(0, K)], didx[bi6],
                              isemd[bi6]).wait()
        pltpu.async_copy(x_hbm.at[sidx[bi6]], xr[b4], gsem[b4])
        pltpu.async_copy(ea_hbm.at[pl.ds(off, K), :], er[b6], esem[b6])

    def wait_scatter(b6, bi6):
        pltpu.make_async_copy(er[b6], acc.at[didx[bi6]], ssem[b6]).wait()

    def process(b4, b6, bi6):
        pltpu.make_async_copy(x_hbm.at[sidx[bi6]], xr[b4], gsem[b4]).wait()
        pltpu.make_async_copy(ea_hbm.at[pl.ds(0, K), :], er[b6],
                              esem[b6]).wait()

        def row(i, rcarry):
            shift16 = jnp.full((16,), 16, jnp.int32)
            hi_mask = jnp.full((16,), -65536, jnp.int32)
            for cc in range(D // 32):
                w = xr[b4][i, pl.ds(cc * 16, 16)]
                lo = lax.bitcast_convert_type(
                    lax.shift_left(w, shift16), jnp.float32)
                hi = lax.bitcast_convert_type(
                    lax.bitwise_and(w, hi_mask), jnp.float32)
                sl_lo = pl.ds(cc * 32, 16)
                sl_hi = pl.ds(cc * 32 + 16, 16)
                er[b6][i, sl_lo] = jnp.maximum(lo + er[b6][i, sl_lo], 0.0)
                er[b6][i, sl_hi] = jnp.maximum(hi + er[b6][i, sl_hi], 0.0)
            return rcarry

        lax.fori_loop(0, K, row, 0)
        pltpu.async_copy(er[b6], acc.at[didx[bi6]], ssem[b6], add=True)

    def step(j_off, jpy):
        process(jpy % NX, jpy % NE, jpy % NIr)
        if jpy >= DG:
            wait_scatter((jpy - DG) % NE, (jpy - DG) % NIr)
        if jpy + DG < CHUNKS:
            gather_start((jpy + DG) % NX, (jpy + DG) % NE, (jpy + DG) % NIr,
                         base + (j_off + DG) * K)
        if jpy + DI < CHUNKS:
            idx_start((jpy + DI) % NIr, base + (j_off + DI) * K)

    for j in range(DI):
        idx_start(j % NIr, base + j * K)
    for j in range(DG):
        gather_start(j % NX, j % NE, j % NIr, base + j * K)

    zbuf = er[NE - 1]

    def zrow(i, rcarry):
        zv = jnp.zeros((16,), jnp.float32)
        for cc in range(D // 16):
            zbuf[i, pl.ds(cc * 16, 16)] = zv
        return rcarry

    lax.fori_loop(0, K, zrow, 0)
    for t in range(RPT // K):
        pltpu.sync_copy(zbuf, acc.at[pl.ds(s * RPT + t * K, K)])
    plsc.subcore_barrier()

    for j in range(G):
        step(j, j)

    steady0 = G
    nsteady = ((CHUNKS - DI - steady0) // G) * G
    ngroups = nsteady // G

    def group(t, carry):
        for bi in range(G):
            step(steady0 + t * G + bi, steady0 + bi)
        return carry

    lax.fori_loop(0, ngroups, group, 0)

    for j in range(steady0 + nsteady, CHUNKS):
        step(j, j)

    for j in range(CHUNKS - DG, CHUNKS):
        wait_scatter(j % NE, j % NIr)

    plsc.subcore_barrier()
    pltpu.sync_copy(acc.at[pl.ds(s * RPT, RPT)],
                    out_hbm.at[c, pl.ds(s * RPT, RPT)])


def _dense_body(x_ref, p_ref, w1_ref, b1_ref, w2_ref, b2_ref, o_ref):
    x = x_ref[...]
    h = x + p_ref[0, :N] + p_ref[1, :N]
    h1 = jnp.maximum(
        jnp.dot(h, w1_ref[...], preferred_element_type=jnp.float32)
        + b1_ref[...], 0.0)
    h2 = (jnp.dot(h1, w2_ref[...], preferred_element_type=jnp.float32)
          + b2_ref[...])
    y = x + h2
    mean = jnp.mean(y, axis=0, keepdims=True)
    var = jnp.mean((y - mean) ** 2, axis=0, keepdims=True)
    o_ref[...] = (y - mean) * lax.rsqrt(var + 1e-5)


def kernel(x, edge_index, edge_attr, W1, b1, W2, b2):
    xp = x.reshape(N, D // 32, 2, 16).transpose(0, 1, 3, 2)
    xp = xp.astype(jnp.bfloat16)
    xi = lax.bitcast_convert_type(xp, jnp.int32).reshape(N, DH)
    partials = _sc_aggregate(xi, edge_index.reshape(2 * E), edge_attr)
    out = pl.pallas_call(
        _dense_body,
        out_shape=jax.ShapeDtypeStruct((N, D), jnp.float32),
    )(x, partials, W1, b1.reshape(1, D), W2, b2.reshape(1, D))
    return out


# speedup vs baseline: 1.4051x; 1.4051x over previous
import functools

import jax
import jax.numpy as jnp
from jax import lax
from jax.experimental import pallas as pl
from jax.experimental.pallas import tpu as pltpu
from jax.experimental.pallas import tpu_sc as plsc

N = 10000
E = 320000
D = 128

NC = 2
NS = 16
NW = NC * NS
EPW = E // NW
K = 40
CHUNKS = EPW // K
N_PAD = 10240
RPT = N_PAD // NS

NX = 3
NE = 6
NIr = 6
G = 6
DG = 2
DI = 4
DH = D // 2

_sc_mesh = plsc.VectorSubcoreMesh(core_axis_name="c", subcore_axis_name="s")

_scratch = []
_scratch += [pltpu.VMEM((K,), jnp.int32)] * NIr
_scratch += [pltpu.VMEM((K,), jnp.int32)] * NIr
_scratch += [pltpu.VMEM((K, D), jnp.float32)] * NX
_scratch += [pltpu.VMEM((K, D), jnp.float32)] * NE
_scratch += [pltpu.SemaphoreType.DMA] * (2 * NIr)
_scratch += [pltpu.SemaphoreType.DMA] * NX
_scratch += [pltpu.SemaphoreType.DMA] * (2 * NE)
_scratch += [pltpu.VMEM_SHARED((N_PAD, D), jnp.float32)]


@functools.partial(
    pl.kernel,
    mesh=_sc_mesh,
    out_type=jax.ShapeDtypeStruct((NC, N_PAD, D), jnp.float32),
    scratch_types=_scratch,
)
def _sc_aggregate(x_hbm, ei_hbm, ea_hbm, out_hbm, *refs):
    o = 0
    sidx = list(refs[o:o + NIr]); o += NIr
    didx = list(refs[o:o + NIr]); o += NIr
    xr = list(refs[o:o + NX]); o += NX
    er = list(refs[o:o + NE]); o += NE
    isems = list(refs[o:o + NIr]); o += NIr
    isemd = list(refs[o:o + NIr]); o += NIr
    gsem = list(refs[o:o + NX]); o += NX
    esem = list(refs[o:o + NE]); o += NE
    ssem = list(refs[o:o + NE]); o += NE
    acc = refs[o]

    c = lax.axis_index("c")
    s = lax.axis_index("s")
    wid = c * NS + s
    base = wid * EPW

    def idx_start(bi6, off):
        pltpu.async_copy(ei_hbm.at[pl.ds(off, K)], sidx[bi6], isems[bi6])
        pltpu.async_copy(ei_hbm.at[pl.ds(E + off, K)], didx[bi6], isemd[bi6])

    def gather_start(b4, b6, bi6, off):
        pltpu.make_async_copy(ei_hbm.at[pl.ds(0, K)], sidx[bi6],
                              isems[bi6]).wait()
        pltpu.make_async_copy(ei_hbm.at[pl.ds(0, K)], didx[bi6],
                              isemd[bi6]).wait()
        pltpu.async_copy(x_hbm.at[sidx[bi6]], xr[b4], gsem[b4])
        pltpu.async_copy(ea_hbm.at[pl.ds(off, K), :], er[b6], esem[b6])

    def wait_scatter(b6, bi6):
        pltpu.make_async_copy(er[b6], acc.at[didx[bi6]], ssem[b6]).wait()

    def process(b4, b6, bi6):
        pltpu.make_async_copy(x_hbm.at[sidx[bi6]], xr[b4], gsem[b4]).wait()
        pltpu.make_async_copy(ea_hbm.at[pl.ds(0, K), :], er[b6],
                              esem[b6]).wait()

        def row(i, rcarry):
            for cc in range(D // 16):
                sl = pl.ds(cc * 16, 16)
                er[b6][i, sl] = jnp.maximum(xr[b4][i, sl] + er[b6][i, sl],
                                            0.0)
            return rcarry

        lax.fori_loop(0, K, row, 0)
        pltpu.async_copy(er[b6], acc.at[didx[bi6]], ssem[b6], add=True)

    def step(j_off, jpy):
        process(jpy % NX, jpy % NE, jpy % NIr)
        if jpy >= DG:
            wait_scatter((jpy - DG) % NE, (jpy - DG) % NIr)
        if jpy + DG < CHUNKS:
            gather_start((jpy + DG) % NX, (jpy + DG) % NE, (jpy + DG) % NIr,
                         base + (j_off + DG) * K)
        if jpy + DI < CHUNKS:
            idx_start((jpy + DI) % NIr, base + (j_off + DI) * K)

    for j in range(DI):
        idx_start(j % NIr, base + j * K)
    for j in range(DG):
        gather_start(j % NX, j % NE, j % NIr, base + j * K)

    zbuf = er[NE - 1]

    def zrow(i, rcarry):
        zv = jnp.zeros((16,), jnp.float32)
        for cc in range(D // 16):
            zbuf[i, pl.ds(cc * 16, 16)] = zv
        return rcarry

    lax.fori_loop(0, K, zrow, 0)
    for t in range(RPT // K):
        pltpu.sync_copy(zbuf, acc.at[pl.ds(s * RPT + t * K, K)])
    plsc.subcore_barrier()

    for j in range(G):
        step(j, j)

    steady0 = G
    nsteady = ((CHUNKS - DI - steady0) // G) * G
    ngroups = nsteady // G

    def group(t, carry):
        for bi in range(G):
            step(steady0 + t * G + bi, steady0 + bi)
        return carry

    lax.fori_loop(0, ngroups, group, 0)

    for j in range(steady0 + nsteady, CHUNKS):
        step(j, j)

    for j in range(CHUNKS - DG, CHUNKS):
        wait_scatter(j % NE, j % NIr)

    plsc.subcore_barrier()
    pltpu.sync_copy(acc.at[pl.ds(s * RPT, RPT)],
                    out_hbm.at[c, pl.ds(s * RPT, RPT)])


def _dense_body(x_ref, p_ref, w1_ref, b1_ref, w2_ref, b2_ref, o_ref):
    x = x_ref[...]
    h = x + p_ref[0, :N] + p_ref[1, :N]
    h1 = jnp.maximum(
        jnp.dot(h, w1_ref[...], preferred_element_type=jnp.float32)
        + b1_ref[...], 0.0)
    h2 = (jnp.dot(h1, w2_ref[...], preferred_element_type=jnp.float32)
          + b2_ref[...])
    y = x + h2
    mean = jnp.mean(y, axis=0, keepdims=True)
    var = jnp.mean((y - mean) ** 2, axis=0, keepdims=True)
    o_ref[...] = (y - mean) * lax.rsqrt(var + 1e-5)


def kernel(x, edge_index, edge_attr, W1, b1, W2, b2):
    partials = _sc_aggregate(x, edge_index.reshape(2 * E), edge_attr)
    out = pl.pallas_call(
        _dense_body,
        out_shape=jax.ShapeDtypeStruct((N, D), jnp.float32),
    )(x, partials, W1, b1.reshape(1, D), W2, b2.reshape(1, D))
    return out
